# Initial kernel scaffold; baseline (speedup 1.0000x reference)
import functools

import jax
import jax.numpy as jnp
from jax import lax
from jax.experimental import pallas as pl
from jax.experimental.pallas import tpu as pltpu
from jax.experimental.pallas import tpu_sc as plsc

VOCAB_NUM = 100000
SEG_NUM = 2
MAX_SEQ_LEN = 2048
D_MODEL = 768
BATCH = 4

NC = 2
NS = 16
NW = NC * NS
B_TOTAL = BATCH * MAX_SEQ_LEN
B_PER_W = B_TOTAL // NW
CHUNK = 64
N_CHUNKS = B_PER_W // CHUNK


def _pos_enc_table():
    pos_ids = jnp.arange(0, MAX_SEQ_LEN, 1, dtype=jnp.float32)[:, None]
    div_term = jnp.power(
        10000.0, jnp.arange(0, D_MODEL, 2, dtype=jnp.float32) / D_MODEL)
    pe = jnp.zeros((MAX_SEQ_LEN, D_MODEL), dtype=jnp.float32)
    pe = pe.at[:, ::2].set(jnp.sin(pos_ids / div_term))
    pe = pe.at[:, 1::2].set(jnp.cos(pos_ids / div_term))
    return pe


@functools.partial(
    pl.kernel,
    out_type=jax.ShapeDtypeStruct((B_TOTAL, D_MODEL), jnp.float32),
    mesh=plsc.VectorSubcoreMesh(core_axis_name="c", subcore_axis_name="s"),
    scratch_types=[
        pltpu.VMEM((CHUNK,), jnp.int32),
        pltpu.VMEM((CHUNK,), jnp.int32),
        pltpu.VMEM((CHUNK, D_MODEL), jnp.float32),
        pltpu.SemaphoreType.DMA,
    ],
)
def _emb_kernel(ids_hbm, segs_hbm, tok_hbm, seg_hbm, pos_hbm, out_hbm,
                idx_t, idx_s, buf, sem):
    wid = lax.axis_index("s") * NC + lax.axis_index("c")
    base = wid * B_PER_W
    pos_base = base % MAX_SEQ_LEN
    for k in range(N_CHUNKS):
        row0 = base + k * CHUNK
        pltpu.sync_copy(pos_hbm.at[pl.ds(pos_base + k * CHUNK, CHUNK)], buf)
        pltpu.sync_copy(ids_hbm.at[pl.ds(row0, CHUNK)], idx_t)
        pltpu.sync_copy(segs_hbm.at[pl.ds(row0, CHUNK)], idx_s)
        pltpu.async_copy(tok_hbm.at[idx_t], buf, sem, add=True).wait()
        pltpu.async_copy(seg_hbm.at[idx_s], buf, sem, add=True).wait()
        pltpu.sync_copy(buf, out_hbm.at[pl.ds(row0, CHUNK)])


def kernel(input_ids, seg_ids, masks, token_table, segment_table):
    del masks
    ids_flat = input_ids.reshape(B_TOTAL).astype(jnp.int32)
    segs_flat = seg_ids.reshape(B_TOTAL).astype(jnp.int32)
    pos = _pos_enc_table()
    out = _emb_kernel(ids_flat, segs_flat, token_table, segment_table, pos)
    return out.reshape(BATCH, MAX_SEQ_LEN, D_MODEL)


# speedup vs baseline: 1.0407x; 1.0407x over previous
import functools

import jax
import jax.numpy as jnp
from jax import lax
from jax.experimental import pallas as pl
from jax.experimental.pallas import tpu as pltpu
from jax.experimental.pallas import tpu_sc as plsc

VOCAB_NUM = 100000
SEG_NUM = 2
MAX_SEQ_LEN = 2048
D_MODEL = 768
BATCH = 4

NC = 2
NS = 16
NW = NC * NS
B_TOTAL = BATCH * MAX_SEQ_LEN
B_PER_W = B_TOTAL // NW
CHUNK = 64
N_CHUNKS = B_PER_W // CHUNK
LANES = 16
GROUPS = D_MODEL // LANES


def _pos_enc_table():
    pos_ids = jnp.arange(0, MAX_SEQ_LEN, 1, dtype=jnp.float32)[:, None]
    div_term = jnp.power(
        10000.0, jnp.arange(0, D_MODEL, 2, dtype=jnp.float32) / D_MODEL)
    pe = jnp.zeros((MAX_SEQ_LEN, D_MODEL), dtype=jnp.float32)
    pe = pe.at[:, ::2].set(jnp.sin(pos_ids / div_term))
    pe = pe.at[:, 1::2].set(jnp.cos(pos_ids / div_term))
    return pe


@functools.partial(
    pl.kernel,
    out_type=jax.ShapeDtypeStruct((B_TOTAL, D_MODEL), jnp.float32),
    mesh=plsc.VectorSubcoreMesh(core_axis_name="c", subcore_axis_name="s"),
    scratch_types=[
        pltpu.VMEM((CHUNK,), jnp.int32),
        pltpu.VMEM((CHUNK,), jnp.int32),
        pltpu.VMEM((CHUNK, D_MODEL), jnp.float32),
        pltpu.VMEM((CHUNK, D_MODEL), jnp.float32),
        pltpu.SemaphoreType.DMA,
        pltpu.SemaphoreType.DMA,
    ],
)
def _emb_kernel(ids_hbm, cids_hbm, tok_hbm, comb_hbm, out_hbm,
                idx_t, idx_c, buf_a, buf_b, sem_a, sem_b):
    wid = lax.axis_index("s") * NC + lax.axis_index("c")
    base = wid * B_PER_W
    for k in range(N_CHUNKS):
        row0 = base + k * CHUNK
        pltpu.sync_copy(ids_hbm.at[pl.ds(row0, CHUNK)], idx_t)
        pltpu.sync_copy(cids_hbm.at[pl.ds(row0, CHUNK)], idx_c)
        cp_a = pltpu.async_copy(tok_hbm.at[idx_t], buf_a, sem_a)
        cp_b = pltpu.async_copy(comb_hbm.at[idx_c], buf_b, sem_b)
        cp_a.wait()
        cp_b.wait()

        def add_row(r, _):
            for g in range(GROUPS):
                sl = pl.ds(g * LANES, LANES)
                plsc.addupdate(buf_a.at[r, sl], buf_b[r, sl])
            return 0

        lax.fori_loop(0, CHUNK, add_row, 0)
        pltpu.sync_copy(buf_a, out_hbm.at[pl.ds(row0, CHUNK)])


def kernel(input_ids, seg_ids, masks, token_table, segment_table):
    del masks
    ids_flat = input_ids.reshape(B_TOTAL).astype(jnp.int32)
    comb = (segment_table[:, None, :] + _pos_enc_table()[None, :, :])
    comb = comb.reshape(SEG_NUM * MAX_SEQ_LEN, D_MODEL)
    cids = (seg_ids.astype(jnp.int32) * MAX_SEQ_LEN
            + jnp.arange(MAX_SEQ_LEN, dtype=jnp.int32)[None, :])
    cids_flat = cids.reshape(B_TOTAL)
    out = _emb_kernel(ids_flat, cids_flat, token_table, comb)
    return out.reshape(BATCH, MAX_SEQ_LEN, D_MODEL)


# speedup vs baseline: 1.5828x; 1.5209x over previous
import functools

import jax
import jax.numpy as jnp
import numpy as np
from jax import lax
from jax.experimental import pallas as pl
from jax.experimental.pallas import tpu as pltpu
from jax.experimental.pallas import tpu_sc as plsc

VOCAB_NUM = 100000
SEG_NUM = 2
MAX_SEQ_LEN = 2048
D_MODEL = 768
BATCH = 4

NC = 2
NS = 16
NW = NC * NS
B_TOTAL = BATCH * MAX_SEQ_LEN
B_PER_W = B_TOTAL // NW
CHUNK = 64
N_CHUNKS = B_PER_W // CHUNK
LANES = 16
GROUPS = D_MODEL // LANES


def _pos_enc_table():
    pos_ids = np.arange(0, MAX_SEQ_LEN, 1, dtype=np.float32)[:, None]
    div_term = np.power(
        10000.0, np.arange(0, D_MODEL, 2, dtype=np.float32) / D_MODEL)
    pe = np.zeros((MAX_SEQ_LEN, D_MODEL), dtype=np.float32)
    pe[:, ::2] = np.sin(pos_ids / div_term)
    pe[:, 1::2] = np.cos(pos_ids / div_term)
    return pe


_POS_ENC = _pos_enc_table()
_IOTA_SEQ = np.arange(MAX_SEQ_LEN, dtype=np.int32)


@functools.partial(
    pl.kernel,
    out_type=jax.ShapeDtypeStruct((B_TOTAL, D_MODEL), jnp.float32),
    mesh=plsc.VectorSubcoreMesh(core_axis_name="c", subcore_axis_name="s"),
    scratch_types=[
        pltpu.VMEM((CHUNK,), jnp.int32),
        pltpu.VMEM((CHUNK,), jnp.int32),
        pltpu.VMEM((CHUNK, D_MODEL), jnp.float32),
        pltpu.VMEM((CHUNK, D_MODEL), jnp.float32),
        pltpu.SemaphoreType.DMA,
        pltpu.SemaphoreType.DMA,
    ],
)
def _emb_kernel(ids_hbm, cids_hbm, tok_hbm, comb_hbm, out_hbm,
                idx_t, idx_c, buf_a, buf_b, sem_a, sem_b):
    wid = lax.axis_index("s") * NC + lax.axis_index("c")
    base = wid * B_PER_W
    for k in range(N_CHUNKS):
        row0 = base + k * CHUNK
        pltpu.sync_copy(ids_hbm.at[pl.ds(row0, CHUNK)], idx_t)
        pltpu.sync_copy(cids_hbm.at[pl.ds(row0, CHUNK)], idx_c)
        cp_a = pltpu.async_copy(tok_hbm.at[idx_t], buf_a, sem_a)
        cp_b = pltpu.async_copy(comb_hbm.at[idx_c], buf_b, sem_b)
        cp_a.wait()
        cp_b.wait()

        def add_row(r, _):
            for g in range(GROUPS):
                sl = pl.ds(g * LANES, LANES)
                plsc.addupdate(buf_a.at[r, sl], buf_b[r, sl])
            return 0

        lax.fori_loop(0, CHUNK, add_row, 0)
        pltpu.sync_copy(buf_a, out_hbm.at[pl.ds(row0, CHUNK)])


def kernel(input_ids, seg_ids, masks, token_table, segment_table):
    del masks
    ids_flat = input_ids.reshape(B_TOTAL).astype(jnp.int32)
    comb = (segment_table[:, None, :] + _POS_ENC[None, :, :])
    comb = comb.reshape(SEG_NUM * MAX_SEQ_LEN, D_MODEL)
    cids = seg_ids.astype(jnp.int32) * MAX_SEQ_LEN + _IOTA_SEQ[None, :]
    cids_flat = cids.reshape(B_TOTAL)
    out = _emb_kernel(ids_flat, cids_flat, token_table, comb)
    return out.reshape(BATCH, MAX_SEQ_LEN, D_MODEL)


# speedup vs baseline: 1.8415x; 1.1635x over previous
import functools

import jax
import jax.numpy as jnp
import numpy as np
from jax import lax
from jax.experimental import pallas as pl
from jax.experimental.pallas import tpu as pltpu
from jax.experimental.pallas import tpu_sc as plsc

VOCAB_NUM = 100000
SEG_NUM = 2
MAX_SEQ_LEN = 2048
D_MODEL = 768
BATCH = 4

NC = 2
NS = 16
NW = NC * NS
B_TOTAL = BATCH * MAX_SEQ_LEN
B_PER_W = B_TOTAL // NW
CHUNK = 32
N_CHUNKS = B_PER_W // CHUNK
LANES = 16
GROUPS = D_MODEL // LANES


def _pos_enc_table():
    pos_ids = np.arange(0, MAX_SEQ_LEN, 1, dtype=np.float32)[:, None]
    div_term = np.power(
        10000.0, np.arange(0, D_MODEL, 2, dtype=np.float32) / D_MODEL)
    pe = np.zeros((MAX_SEQ_LEN, D_MODEL), dtype=np.float32)
    pe[:, ::2] = np.sin(pos_ids / div_term)
    pe[:, 1::2] = np.cos(pos_ids / div_term)
    return pe


_POS_ENC = _pos_enc_table()
_IOTA_SEQ = np.arange(MAX_SEQ_LEN, dtype=np.int32)


@functools.partial(
    pl.kernel,
    out_type=jax.ShapeDtypeStruct((B_TOTAL, D_MODEL), jnp.float32),
    mesh=plsc.VectorSubcoreMesh(core_axis_name="c", subcore_axis_name="s"),
    scratch_types=[
        [pltpu.VMEM((CHUNK,), jnp.int32)] * 2,
        [pltpu.VMEM((CHUNK,), jnp.int32)] * 2,
        [pltpu.VMEM((CHUNK, D_MODEL), jnp.float32)] * 2,
        [pltpu.VMEM((CHUNK, D_MODEL), jnp.float32)] * 2,
        [pltpu.SemaphoreType.DMA] * 2,
        [pltpu.SemaphoreType.DMA] * 2,
        [pltpu.SemaphoreType.DMA] * 2,
    ],
)
def _emb_kernel(ids_hbm, cids_hbm, tok_hbm, comb_hbm, out_hbm,
                idx_t, idx_c, buf_a, buf_b, sem_a, sem_b, sem_o):
    wid = lax.axis_index("s") * NC + lax.axis_index("c")
    base = wid * B_PER_W

    def issue(k, s):
        row0 = base + k * CHUNK
        pltpu.sync_copy(ids_hbm.at[pl.ds(row0, CHUNK)], idx_t[s])
        pltpu.sync_copy(cids_hbm.at[pl.ds(row0, CHUNK)], idx_c[s])
        cp_a = pltpu.async_copy(tok_hbm.at[idx_t[s]], buf_a[s], sem_a[s])
        cp_b = pltpu.async_copy(comb_hbm.at[idx_c[s]], buf_b[s], sem_b[s])
        return cp_a, cp_b

    gathers = [None, None]
    stores = [None, None]
    gathers[0] = issue(0, 0)
    for k in range(N_CHUNKS):
        s = k % 2
        n = (k + 1) % 2
        if k + 1 < N_CHUNKS:
            if stores[n] is not None:
                stores[n].wait()
            gathers[n] = issue(k + 1, n)
        cp_a, cp_b = gathers[s]
        cp_a.wait()
        cp_b.wait()

        def add_row(r, _, s=s):
            for g in range(GROUPS):
                sl = pl.ds(g * LANES, LANES)
                plsc.addupdate(buf_a[s].at[r, sl], buf_b[s][r, sl])
            return 0

        lax.fori_loop(0, CHUNK, add_row, 0)
        row0 = base + k * CHUNK
        stores[s] = pltpu.async_copy(
            buf_a[s], out_hbm.at[pl.ds(row0, CHUNK)], sem_o[s])
    stores[0].wait()
    stores[1].wait()


def kernel(input_ids, seg_ids, masks, token_table, segment_table):
    del masks
    ids_flat = input_ids.reshape(B_TOTAL).astype(jnp.int32)
    comb = (segment_table[:, None, :] + _POS_ENC[None, :, :])
    comb = comb.reshape(SEG_NUM * MAX_SEQ_LEN, D_MODEL)
    cids = seg_ids.astype(jnp.int32) * MAX_SEQ_LEN + _IOTA_SEQ[None, :]
    cids_flat = cids.reshape(B_TOTAL)
    out = _emb_kernel(ids_flat, cids_flat, token_table, comb)
    return out.reshape(BATCH, MAX_SEQ_LEN, D_MODEL)


# speedup vs baseline: 1.9056x; 1.0348x over previous
import functools

import jax
import jax.numpy as jnp
import numpy as np
from jax import lax
from jax.experimental import pallas as pl
from jax.experimental.pallas import tpu as pltpu
from jax.experimental.pallas import tpu_sc as plsc

VOCAB_NUM = 100000
SEG_NUM = 2
MAX_SEQ_LEN = 2048
D_MODEL = 768
BATCH = 4

NC = 2
NS = 16
NW = NC * NS
B_TOTAL = BATCH * MAX_SEQ_LEN
B_PER_W = B_TOTAL // NW
W_PER_B = MAX_SEQ_LEN // B_PER_W
CHUNK = 32
N_CHUNKS = B_PER_W // CHUNK
LANES = 16
GROUPS = D_MODEL // LANES


def _pos_enc_table():
    pos_ids = np.arange(0, MAX_SEQ_LEN, 1, dtype=np.float32)[:, None]
    div_term = np.power(
        10000.0, np.arange(0, D_MODEL, 2, dtype=np.float32) / D_MODEL)
    pe = np.zeros((MAX_SEQ_LEN, D_MODEL), dtype=np.float32)
    pe[:, ::2] = np.sin(pos_ids / div_term)
    pe[:, 1::2] = np.cos(pos_ids / div_term)
    return pe


_POS_ENC = _pos_enc_table()


@functools.partial(
    pl.kernel,
    out_type=jax.ShapeDtypeStruct((BATCH, MAX_SEQ_LEN, D_MODEL), jnp.float32),
    mesh=plsc.VectorSubcoreMesh(core_axis_name="c", subcore_axis_name="s"),
    scratch_types=[
        [pltpu.VMEM((CHUNK,), jnp.int32)] * 2,
        [pltpu.VMEM((CHUNK,), jnp.int32)] * 2,
        [pltpu.VMEM((CHUNK, D_MODEL), jnp.float32)] * 2,
        [pltpu.VMEM((CHUNK, D_MODEL), jnp.float32)] * 2,
        [pltpu.SemaphoreType.DMA] * 2,
        [pltpu.SemaphoreType.DMA] * 2,
        [pltpu.SemaphoreType.DMA] * 2,
    ],
)
def _emb_kernel(ids_hbm, segs_hbm, tok_hbm, comb_hbm, out_hbm,
                idx_t, idx_c, buf_a, buf_b, sem_a, sem_b, sem_o):
    wid = lax.axis_index("s") * NC + lax.axis_index("c")
    b = wid // W_PER_B
    pos_base = (wid % W_PER_B) * B_PER_W

    def issue(k, s):
        p0 = pos_base + k * CHUNK
        pltpu.sync_copy(ids_hbm.at[b, pl.ds(p0, CHUNK)], idx_t[s])
        pltpu.sync_copy(segs_hbm.at[b, pl.ds(p0, CHUNK)], idx_c[s])
        for g in range(CHUNK // LANES):
            sl = pl.ds(g * LANES, LANES)
            seg = idx_c[s][sl]
            iota = lax.iota(jnp.int32, LANES)
            idx_c[s][sl] = seg * MAX_SEQ_LEN + (p0 + g * LANES) + iota
        cp_a = pltpu.async_copy(tok_hbm.at[idx_t[s]], buf_a[s], sem_a[s])
        cp_b = pltpu.async_copy(comb_hbm.at[idx_c[s]], buf_b[s], sem_b[s])
        return cp_a, cp_b

    gathers = [None, None]
    stores = [None, None]
    gathers[0] = issue(0, 0)
    for k in range(N_CHUNKS):
        s = k % 2
        n = (k + 1) % 2
        if k + 1 < N_CHUNKS:
            if stores[n] is not None:
                stores[n].wait()
            gathers[n] = issue(k + 1, n)
        cp_a, cp_b = gathers[s]
        cp_a.wait()
        cp_b.wait()

        def add_row(r, _, s=s):
            for g in range(GROUPS):
                sl = pl.ds(g * LANES, LANES)
                plsc.addupdate(buf_a[s].at[r, sl], buf_b[s][r, sl])
            return 0

        lax.fori_loop(0, CHUNK, add_row, 0)
        stores[s] = pltpu.async_copy(
            buf_a[s], out_hbm.at[b, pl.ds(pos_base + k * CHUNK, CHUNK)],
            sem_o[s])
    stores[0].wait()
    stores[1].wait()


def kernel(input_ids, seg_ids, masks, token_table, segment_table):
    del masks
    comb = (segment_table[:, None, :] + _POS_ENC[None, :, :])
    comb = comb.reshape(SEG_NUM * MAX_SEQ_LEN, D_MODEL)
    return _emb_kernel(input_ids.astype(jnp.int32),
                       seg_ids.astype(jnp.int32), token_table, comb)
